# Initial kernel scaffold; baseline (speedup 1.0000x reference)
import jax
import jax.numpy as jnp
from jax.experimental import pallas as pl


def kernel(x, w1, b1, w2, b2, wr, br):
    raise NotImplementedError("write your pallas kernel here")


# speedup vs baseline: 1.7103x; 1.7103x over previous
import functools

import jax
import jax.numpy as jnp
from jax.experimental import pallas as pl
from jax.experimental.pallas import tpu as pltpu


def _fused_body(x_ref, w1_ref, b1_ref, w2_ref, b2_ref, wr_ref, br_ref,
                out_ref, p_ref, *, n_h):
    h = pl.program_id(1)

    @pl.when(h == 0)
    def _router():
        logits = jnp.dot(x_ref[...], wr_ref[...],
                         preferred_element_type=jnp.float32) + br_ref[...]
        mx = jnp.max(logits, axis=-1, keepdims=True)
        p_ref[...] = 1.0 / jnp.sum(jnp.exp(logits - mx), axis=-1,
                                   keepdims=True)
        out_ref[...] = jnp.zeros_like(out_ref)

    hidden = jnp.maximum(
        jnp.dot(x_ref[...], w1_ref[...],
                preferred_element_type=jnp.float32) + b1_ref[...], 0.0)
    out_ref[...] += jnp.dot(hidden, w2_ref[...],
                            preferred_element_type=jnp.float32)

    @pl.when(h == n_h - 1)
    def _finish():
        out_ref[...] = (out_ref[...] + b2_ref[...]) * p_ref[...]


def kernel(x, w1, b1, w2, b2, wr, br):
    batch, seq, d = x.shape
    t = batch * seq
    hdim = w1.shape[1]
    e = wr.shape[1]

    tokens = x.reshape(t, d)

    ep = 128
    wr_pad = jnp.concatenate(
        [wr, jnp.zeros((d, ep - e), wr.dtype)], axis=1)
    br_pad = jnp.concatenate(
        [br, jnp.full((ep - e,), -1e30, br.dtype)], axis=0).reshape(1, ep)
    b1r = b1.reshape(1, hdim)
    b2r = b2.reshape(1, d)

    bm = min(1024, t)
    bh = min(512, hdim)
    n_m = t // bm
    n_h = hdim // bh

    out = pl.pallas_call(
        functools.partial(_fused_body, n_h=n_h),
        grid=(n_m, n_h),
        in_specs=[
            pl.BlockSpec((bm, d), lambda m, h: (m, 0)),
            pl.BlockSpec((d, bh), lambda m, h: (0, h)),
            pl.BlockSpec((1, bh), lambda m, h: (0, h)),
            pl.BlockSpec((bh, d), lambda m, h: (h, 0)),
            pl.BlockSpec((1, d), lambda m, h: (0, 0)),
            pl.BlockSpec((d, ep), lambda m, h: (0, 0)),
            pl.BlockSpec((1, ep), lambda m, h: (0, 0)),
        ],
        out_specs=pl.BlockSpec((bm, d), lambda m, h: (m, 0)),
        out_shape=jax.ShapeDtypeStruct((t, d), jnp.float32),
        scratch_shapes=[pltpu.VMEM((bm, 1), jnp.float32)],
        compiler_params=pltpu.CompilerParams(
            dimension_semantics=("parallel", "arbitrary")),
    )(tokens, w1, b1r, w2, b2r, wr_pad, br_pad)

    return out.reshape(batch, seq, d)
